# baseline (speedup 1.0000x reference)
import functools

import jax
import jax.numpy as jnp
import numpy as np
from jax import lax
from jax.experimental import pallas as pl
from jax.experimental.pallas import tpu as pltpu
from jax.experimental.pallas import tpu_sc as plsc

_MBASE = int(np.float32(1e10).view(np.int32))
_IMAX = 2147483647

_FPS_NUM = 512
_GROUP_NUM = 32
_RADIUS = 0.2
_BQ_BLOCK = 128


def _fps_body(x_ref, y_ref, z_ref, c0x_ref, c0y_ref, c0z_ref,
              cx_ref, cy_ref, cz_ref, dist_ref):
    B, N = x_ref.shape
    dist_ref[...] = jnp.full((B, N), 1e10, dtype=jnp.float32)
    iota = jax.lax.broadcasted_iota(jnp.int32, (B, N), 1)
    col = jax.lax.broadcasted_iota(jnp.int32, cx_ref.shape, 1)

    def body(i, carry):
        cx, cy, cz = carry
        cx_ref[...] = jnp.where(col == i, cx, cx_ref[...])
        cy_ref[...] = jnp.where(col == i, cy, cy_ref[...])
        cz_ref[...] = jnp.where(col == i, cz, cz_ref[...])
        dx = x_ref[...] - cx
        dy = y_ref[...] - cy
        dz = z_ref[...] - cz
        dd = dx * dx + dy * dy + dz * dz
        dold = dist_ref[...]
        dnew = jnp.where(dd < dold, dd, dold)
        dist_ref[...] = dnew
        m = jnp.max(dnew, axis=1, keepdims=True)
        tie = jnp.where(dnew == m, iota, N)
        amin = jnp.min(tie, axis=1, keepdims=True)
        em = iota == amin
        ncx = jnp.sum(jnp.where(em, x_ref[...], 0.0), axis=1, keepdims=True)
        ncy = jnp.sum(jnp.where(em, y_ref[...], 0.0), axis=1, keepdims=True)
        ncz = jnp.sum(jnp.where(em, z_ref[...], 0.0), axis=1, keepdims=True)
        return (ncx, ncy, ncz)

    jax.lax.fori_loop(0, _FPS_NUM, body,
                      (c0x_ref[...], c0y_ref[...], c0z_ref[...]))


def _sc_bq_body(x_ref, y_ref, z_ref, cx_ref, cy_ref, cz_ref,
                rx_ref, ry_ref, rz_ref,
                xv, yv, zv, cxv, cyv, czv, rxv, ryv, rzv):
    NC = 2
    wid = lax.axis_index("s") * NC + lax.axis_index("c")
    b = wid // 4
    CW = 128
    GW = CW * _GROUP_NUM
    N = x_ref.shape[1]
    pltpu.sync_copy(x_ref.at[b], xv)
    pltpu.sync_copy(y_ref.at[b], yv)
    pltpu.sync_copy(z_ref.at[b], zv)
    pltpu.sync_copy(cx_ref.at[pl.ds(wid * CW, CW)], cxv)
    pltpu.sync_copy(cy_ref.at[pl.ds(wid * CW, CW)], cyv)
    pltpu.sync_copy(cz_ref.at[pl.ds(wid * CW, CW)], czv)

    lanes = lax.iota(jnp.int32, 16)
    imax = jnp.int32(_IMAX)
    r2 = jnp.float32(_RADIUS ** 2)

    def per_centroid(f, _):
        fv = jnp.full((16,), f, dtype=jnp.int32)
        cxb = plsc.load_gather(cxv, [fv])
        cyb = plsc.load_gather(cyv, [fv])
        czb = plsc.load_gather(czv, [fv])

        def scan_vreg(j, carry):
            pk0, pk1, pi0, pi1, thr = carry
            o = j * 16
            dx = xv[pl.ds(o, 16)] - cxb
            dy = yv[pl.ds(o, 16)] - cyb
            dz = zv[pl.ds(o, 16)] - czb
            dd = dx * dx + dy * dy + dz * dz
            gidx = o + lanes
            kv = jnp.where(dd <= r2, plsc.bitcast(dd, jnp.int32),
                           _MBASE + gidx)

            def w_cond(st):
                _, _, _, _, thr_c, rem = st
                return jnp.any((kv < thr_c) & rem)

            def w_step(st):
                pk0, pk1, pi0, pi1, _, rem = st
                cand = (kv < st[4]) & rem
                l = plsc.all_reduce_ffs(cand)
                onel = lanes == l
                ck = jnp.min(jnp.where(onel, kv, imax))
                ci = jnp.min(jnp.where(onel, gidx, imax))
                mk = jnp.maximum(jnp.max(pk0), jnp.max(pk1))
                mi = jnp.maximum(jnp.max(jnp.where(pk0 == mk, pi0, -1)),
                                 jnp.max(jnp.where(pk1 == mk, pi1, -1)))
                oh0 = (pk0 == mk) & (pi0 == mi)
                oh1 = (pk1 == mk) & (pi1 == mi)
                pk0 = jnp.where(oh0, ck, pk0)
                pk1 = jnp.where(oh1, ck, pk1)
                pi0 = jnp.where(oh0, ci, pi0)
                pi1 = jnp.where(oh1, ci, pi1)
                thr_n = jnp.maximum(jnp.max(pk0), jnp.max(pk1))
                return (pk0, pk1, pi0, pi1, thr_n, rem & (~onel))

            st = lax.while_loop(
                w_cond, w_step,
                (pk0, pk1, pi0, pi1, thr, jnp.full((16,), True)))
            return st[:5]

        pool0 = (
            jnp.full((16,), imax, dtype=jnp.int32),
            jnp.full((16,), imax, dtype=jnp.int32),
            (1 << 24) + lanes,
            (1 << 24) + 16 + lanes,
            imax,
        )
        pk0, pk1, pi0, pi1, _ = lax.fori_loop(0, N // 16, scan_vreg, pool0)

        def extract(k, st):
            pk0, pk1, pi0, pi1, iv0, iv1 = st
            mk = jnp.minimum(jnp.min(pk0), jnp.min(pk1))
            mi = jnp.minimum(jnp.min(jnp.where(pk0 == mk, pi0, imax)),
                             jnp.min(jnp.where(pk1 == mk, pi1, imax)))
            oh0 = (pk0 == mk) & (pi0 == mi)
            oh1 = (pk1 == mk) & (pi1 == mi)
            pk0 = jnp.where(oh0, imax, pk0)
            pk1 = jnp.where(oh1, imax, pk1)
            iv0 = jnp.where(lanes == k, mi, iv0)
            iv1 = jnp.where(lanes == (k - 16), mi, iv1)
            return (pk0, pk1, pi0, pi1, iv0, iv1)

        zero16 = jnp.zeros((16,), dtype=jnp.int32)
        _, _, _, _, iv0, iv1 = lax.fori_loop(
            0, _GROUP_NUM, extract, (pk0, pk1, pi0, pi1, zero16, zero16))

        base = f * _GROUP_NUM
        rxv[pl.ds(base, 16)] = plsc.load_gather(xv, [iv0]) - cxb
        ryv[pl.ds(base, 16)] = plsc.load_gather(yv, [iv0]) - cyb
        rzv[pl.ds(base, 16)] = plsc.load_gather(zv, [iv0]) - czb
        rxv[pl.ds(base + 16, 16)] = plsc.load_gather(xv, [iv1]) - cxb
        ryv[pl.ds(base + 16, 16)] = plsc.load_gather(yv, [iv1]) - cyb
        rzv[pl.ds(base + 16, 16)] = plsc.load_gather(zv, [iv1]) - czb
        return 0

    lax.fori_loop(0, CW, per_centroid, 0)
    pltpu.sync_copy(rxv, rx_ref.at[pl.ds(wid * GW, GW)])
    pltpu.sync_copy(ryv, ry_ref.at[pl.ds(wid * GW, GW)])
    pltpu.sync_copy(rzv, rz_ref.at[pl.ds(wid * GW, GW)])


@jax.jit
def kernel(xyz):
    B, N, _ = xyz.shape
    F, G, C = _FPS_NUM, _GROUP_NUM, _BQ_BLOCK
    xyz_t = jnp.transpose(xyz, (2, 0, 1))
    X, Y, Z = xyz_t[0], xyz_t[1], xyz_t[2]

    f0 = jax.random.randint(jax.random.key(1), (B,), 0, N, dtype=jnp.int32)
    c0 = xyz[jnp.arange(B), f0]
    c0x, c0y, c0z = c0[:, 0:1], c0[:, 1:2], c0[:, 2:3]

    cxs = jax.ShapeDtypeStruct((B, F), jnp.float32)
    CX, CY, CZ = pl.pallas_call(
        _fps_body,
        out_shape=(cxs, cxs, cxs),
        scratch_shapes=[pltpu.VMEM((B, N), jnp.float32)],
    )(X, Y, Z, c0x, c0y, c0z)

    mesh = plsc.VectorSubcoreMesh(core_axis_name="c", subcore_axis_name="s")
    GW = (F // 4) * G
    relf = jax.ShapeDtypeStruct((B * F * G,), jnp.float32)
    RX, RY, RZ = pl.kernel(
        _sc_bq_body,
        mesh=mesh,
        out_type=(relf, relf, relf),
        compiler_params=pltpu.CompilerParams(needs_layout_passes=False),
        scratch_types=[
            pltpu.VMEM((N,), jnp.float32),
            pltpu.VMEM((N,), jnp.float32),
            pltpu.VMEM((N,), jnp.float32),
            pltpu.VMEM((F // 4,), jnp.float32),
            pltpu.VMEM((F // 4,), jnp.float32),
            pltpu.VMEM((F // 4,), jnp.float32),
            pltpu.VMEM((GW,), jnp.float32),
            pltpu.VMEM((GW,), jnp.float32),
            pltpu.VMEM((GW,), jnp.float32),
        ],
    )(X, Y, Z, CX.reshape(-1), CY.reshape(-1), CZ.reshape(-1))

    cent = jnp.stack([CX, CY, CZ], axis=-1)
    rel = jnp.stack([RX.reshape(B, F, G), RY.reshape(B, F, G),
                     RZ.reshape(B, F, G)], axis=-1)
    combined = jnp.concatenate([cent[:, :, None, :], rel], axis=2)
    return (combined, cent)


# speedup vs baseline: 2.2432x; 2.2432x over previous
import functools

import jax
import jax.numpy as jnp
import numpy as np
from jax import lax
from jax.experimental import pallas as pl
from jax.experimental.pallas import tpu as pltpu
from jax.experimental.pallas import tpu_sc as plsc

_MBASE = int(np.float32(1e10).view(np.int32))
_IMAX = 2147483647

_FPS_NUM = 512
_GROUP_NUM = 32
_RADIUS = 0.2
_BQ_BLOCK = 128


def _fps_body(x_ref, y_ref, z_ref, c0x_ref, c0y_ref, c0z_ref,
              cx_ref, cy_ref, cz_ref, dist_ref):
    B, N = x_ref.shape
    dist_ref[...] = jnp.full((B, N), 1e10, dtype=jnp.float32)
    iota = jax.lax.broadcasted_iota(jnp.int32, (B, N), 1)
    col = jax.lax.broadcasted_iota(jnp.int32, cx_ref.shape, 1)

    def body(i, carry):
        cx, cy, cz = carry
        cx_ref[...] = jnp.where(col == i, cx, cx_ref[...])
        cy_ref[...] = jnp.where(col == i, cy, cy_ref[...])
        cz_ref[...] = jnp.where(col == i, cz, cz_ref[...])
        dx = x_ref[...] - cx
        dy = y_ref[...] - cy
        dz = z_ref[...] - cz
        dd = dx * dx + dy * dy + dz * dz
        dold = dist_ref[...]
        dnew = jnp.where(dd < dold, dd, dold)
        dist_ref[...] = dnew
        m = jnp.max(dnew, axis=1, keepdims=True)
        tie = jnp.where(dnew == m, iota, N)
        amin = jnp.min(tie, axis=1, keepdims=True)
        em = iota == amin
        ncx = jnp.sum(jnp.where(em, x_ref[...], 0.0), axis=1, keepdims=True)
        ncy = jnp.sum(jnp.where(em, y_ref[...], 0.0), axis=1, keepdims=True)
        ncz = jnp.sum(jnp.where(em, z_ref[...], 0.0), axis=1, keepdims=True)
        return (ncx, ncy, ncz)

    jax.lax.fori_loop(0, _FPS_NUM, body,
                      (c0x_ref[...], c0y_ref[...], c0z_ref[...]))


def _sc_bq_body(x_ref, y_ref, z_ref, cx_ref, cy_ref, cz_ref,
                rx_ref, ry_ref, rz_ref,
                xv, yv, zv, cxv, cyv, czv, rxv, ryv, rzv):
    NC = 2
    wid = lax.axis_index("s") * NC + lax.axis_index("c")
    b = wid // 4
    CW = 128
    GW = CW * _GROUP_NUM
    N = x_ref.shape[1]
    pltpu.sync_copy(x_ref.at[b], xv)
    pltpu.sync_copy(y_ref.at[b], yv)
    pltpu.sync_copy(z_ref.at[b], zv)
    pltpu.sync_copy(cx_ref.at[pl.ds(wid * CW, CW)], cxv)
    pltpu.sync_copy(cy_ref.at[pl.ds(wid * CW, CW)], cyv)
    pltpu.sync_copy(cz_ref.at[pl.ds(wid * CW, CW)], czv)

    lanes = lax.iota(jnp.int32, 16)
    imax = jnp.int32(_IMAX)
    r2 = jnp.float32(_RADIUS ** 2)

    def per_centroid(f, _):
        fv = jnp.full((16,), f, dtype=jnp.int32)
        cxb = plsc.load_gather(cxv, [fv])
        cyb = plsc.load_gather(cyv, [fv])
        czb = plsc.load_gather(czv, [fv])

        def insert_candidates(kv, gidx, pool):
            def w_cond(st):
                _, _, _, _, thr_c, rem = st
                return jnp.any((kv < thr_c) & rem)

            def w_step(st):
                pk0, pk1, pi0, pi1, _, rem = st
                cand = (kv < st[4]) & rem
                l = plsc.all_reduce_ffs(cand)
                onel = lanes == l
                ck = jnp.min(jnp.where(onel, kv, imax))
                ci = jnp.min(jnp.where(onel, gidx, imax))
                mk = jnp.maximum(jnp.max(pk0), jnp.max(pk1))
                mi = jnp.maximum(jnp.max(jnp.where(pk0 == mk, pi0, -1)),
                                 jnp.max(jnp.where(pk1 == mk, pi1, -1)))
                oh0 = (pk0 == mk) & (pi0 == mi)
                oh1 = (pk1 == mk) & (pi1 == mi)
                pk0 = jnp.where(oh0, ck, pk0)
                pk1 = jnp.where(oh1, ck, pk1)
                pi0 = jnp.where(oh0, ci, pi0)
                pi1 = jnp.where(oh1, ci, pi1)
                thr_n = jnp.maximum(jnp.max(pk0), jnp.max(pk1))
                return (pk0, pk1, pi0, pi1, thr_n, rem & (~onel))

            st = lax.while_loop(w_cond, w_step,
                                pool + (jnp.full((16,), True),))
            return st[:5]

        UNROLL = 4

        def scan_vreg(j, carry):
            pool = carry
            o = j * (16 * UNROLL)
            kvs = []
            for u in range(UNROLL):
                ou = o + u * 16
                dx = xv[pl.ds(ou, 16)] - cxb
                dy = yv[pl.ds(ou, 16)] - cyb
                dz = zv[pl.ds(ou, 16)] - czb
                dd = dx * dx + dy * dy + dz * dz
                kvs.append(jnp.where(dd <= r2, plsc.bitcast(dd, jnp.int32),
                                     _MBASE + (ou + lanes)))
            thr_c = pool[4]
            hit = (kvs[0] < thr_c) | (kvs[1] < thr_c)
            hit = hit | (kvs[2] < thr_c) | (kvs[3] < thr_c)

            def do_insert(pool):
                for u in range(UNROLL):
                    pool = insert_candidates(kvs[u], o + u * 16 + lanes, pool)
                return pool

            return lax.cond(jnp.any(hit), do_insert, lambda p: p, pool)

        pool0 = (
            jnp.full((16,), imax, dtype=jnp.int32),
            jnp.full((16,), imax, dtype=jnp.int32),
            (1 << 24) + lanes,
            (1 << 24) + 16 + lanes,
            imax,
        )
        pk0, pk1, pi0, pi1, _ = lax.fori_loop(0, N // (16 * UNROLL),
                                              scan_vreg, pool0)

        def extract(k, st):
            pk0, pk1, pi0, pi1, iv0, iv1 = st
            mk = jnp.minimum(jnp.min(pk0), jnp.min(pk1))
            mi = jnp.minimum(jnp.min(jnp.where(pk0 == mk, pi0, imax)),
                             jnp.min(jnp.where(pk1 == mk, pi1, imax)))
            oh0 = (pk0 == mk) & (pi0 == mi)
            oh1 = (pk1 == mk) & (pi1 == mi)
            pk0 = jnp.where(oh0, imax, pk0)
            pk1 = jnp.where(oh1, imax, pk1)
            iv0 = jnp.where(lanes == k, mi, iv0)
            iv1 = jnp.where(lanes == (k - 16), mi, iv1)
            return (pk0, pk1, pi0, pi1, iv0, iv1)

        zero16 = jnp.zeros((16,), dtype=jnp.int32)
        _, _, _, _, iv0, iv1 = lax.fori_loop(
            0, _GROUP_NUM, extract, (pk0, pk1, pi0, pi1, zero16, zero16))

        base = f * _GROUP_NUM
        rxv[pl.ds(base, 16)] = plsc.load_gather(xv, [iv0]) - cxb
        ryv[pl.ds(base, 16)] = plsc.load_gather(yv, [iv0]) - cyb
        rzv[pl.ds(base, 16)] = plsc.load_gather(zv, [iv0]) - czb
        rxv[pl.ds(base + 16, 16)] = plsc.load_gather(xv, [iv1]) - cxb
        ryv[pl.ds(base + 16, 16)] = plsc.load_gather(yv, [iv1]) - cyb
        rzv[pl.ds(base + 16, 16)] = plsc.load_gather(zv, [iv1]) - czb
        return 0

    lax.fori_loop(0, CW, per_centroid, 0)
    pltpu.sync_copy(rxv, rx_ref.at[pl.ds(wid * GW, GW)])
    pltpu.sync_copy(ryv, ry_ref.at[pl.ds(wid * GW, GW)])
    pltpu.sync_copy(rzv, rz_ref.at[pl.ds(wid * GW, GW)])


@jax.jit
def kernel(xyz):
    B, N, _ = xyz.shape
    F, G, C = _FPS_NUM, _GROUP_NUM, _BQ_BLOCK
    xyz_t = jnp.transpose(xyz, (2, 0, 1))
    X, Y, Z = xyz_t[0], xyz_t[1], xyz_t[2]

    f0 = jax.random.randint(jax.random.key(1), (B,), 0, N, dtype=jnp.int32)
    c0 = xyz[jnp.arange(B), f0]
    c0x, c0y, c0z = c0[:, 0:1], c0[:, 1:2], c0[:, 2:3]

    cxs = jax.ShapeDtypeStruct((B, F), jnp.float32)
    CX, CY, CZ = pl.pallas_call(
        _fps_body,
        out_shape=(cxs, cxs, cxs),
        scratch_shapes=[pltpu.VMEM((B, N), jnp.float32)],
    )(X, Y, Z, c0x, c0y, c0z)

    mesh = plsc.VectorSubcoreMesh(core_axis_name="c", subcore_axis_name="s")
    GW = (F // 4) * G
    relf = jax.ShapeDtypeStruct((B * F * G,), jnp.float32)
    RX, RY, RZ = pl.kernel(
        _sc_bq_body,
        mesh=mesh,
        out_type=(relf, relf, relf),
        compiler_params=pltpu.CompilerParams(needs_layout_passes=False),
        scratch_types=[
            pltpu.VMEM((N,), jnp.float32),
            pltpu.VMEM((N,), jnp.float32),
            pltpu.VMEM((N,), jnp.float32),
            pltpu.VMEM((F // 4,), jnp.float32),
            pltpu.VMEM((F // 4,), jnp.float32),
            pltpu.VMEM((F // 4,), jnp.float32),
            pltpu.VMEM((GW,), jnp.float32),
            pltpu.VMEM((GW,), jnp.float32),
            pltpu.VMEM((GW,), jnp.float32),
        ],
    )(X, Y, Z, CX.reshape(-1), CY.reshape(-1), CZ.reshape(-1))

    cent = jnp.stack([CX, CY, CZ], axis=-1)
    rel = jnp.stack([RX.reshape(B, F, G), RY.reshape(B, F, G),
                     RZ.reshape(B, F, G)], axis=-1)
    combined = jnp.concatenate([cent[:, :, None, :], rel], axis=2)
    return (combined, cent)


# speedup vs baseline: 2.6731x; 1.1917x over previous
import functools

import jax
import jax.numpy as jnp
import numpy as np
from jax import lax
from jax.experimental import pallas as pl
from jax.experimental.pallas import tpu as pltpu
from jax.experimental.pallas import tpu_sc as plsc

_MBASE = int(np.float32(1e10).view(np.int32))
_IMAX = 2147483647

_FPS_NUM = 512
_GROUP_NUM = 32
_RADIUS = 0.2
_BQ_BLOCK = 128


def _fps_body(x_ref, y_ref, z_ref, c0x_ref, c0y_ref, c0z_ref,
              cx_ref, cy_ref, cz_ref, dist_ref):
    B, N = x_ref.shape
    dist_ref[...] = jnp.full((B, N), 1e10, dtype=jnp.float32)
    iota = jax.lax.broadcasted_iota(jnp.int32, (B, N), 1)
    col = jax.lax.broadcasted_iota(jnp.int32, cx_ref.shape, 1)

    def body(i, carry):
        cx, cy, cz = carry
        cx_ref[...] = jnp.where(col == i, cx, cx_ref[...])
        cy_ref[...] = jnp.where(col == i, cy, cy_ref[...])
        cz_ref[...] = jnp.where(col == i, cz, cz_ref[...])
        dx = x_ref[...] - cx
        dy = y_ref[...] - cy
        dz = z_ref[...] - cz
        dd = dx * dx + dy * dy + dz * dz
        dold = dist_ref[...]
        dnew = jnp.where(dd < dold, dd, dold)
        dist_ref[...] = dnew
        m = jnp.max(dnew, axis=1, keepdims=True)
        tie = jnp.where(dnew == m, iota, N)
        amin = jnp.min(tie, axis=1, keepdims=True)
        em = iota == amin
        ncx = jnp.sum(jnp.where(em, x_ref[...], 0.0), axis=1, keepdims=True)
        ncy = jnp.sum(jnp.where(em, y_ref[...], 0.0), axis=1, keepdims=True)
        ncz = jnp.sum(jnp.where(em, z_ref[...], 0.0), axis=1, keepdims=True)
        return (ncx, ncy, ncz)

    jax.lax.fori_loop(0, _FPS_NUM, body,
                      (c0x_ref[...], c0y_ref[...], c0z_ref[...]))


def _sc_bq_body(x_ref, y_ref, z_ref, cx_ref, cy_ref, cz_ref,
                rx_ref, ry_ref, rz_ref,
                xv, yv, zv, cxv, cyv, czv, rxv, ryv, rzv):
    NC = 2
    wid = lax.axis_index("s") * NC + lax.axis_index("c")
    b = wid // 4
    CW = 128
    GW = CW * _GROUP_NUM
    N = x_ref.shape[1]
    pltpu.sync_copy(x_ref.at[b], xv)
    pltpu.sync_copy(y_ref.at[b], yv)
    pltpu.sync_copy(z_ref.at[b], zv)
    pltpu.sync_copy(cx_ref.at[pl.ds(wid * CW, CW)], cxv)
    pltpu.sync_copy(cy_ref.at[pl.ds(wid * CW, CW)], cyv)
    pltpu.sync_copy(cz_ref.at[pl.ds(wid * CW, CW)], czv)

    lanes = lax.iota(jnp.int32, 16)
    imax = jnp.int32(_IMAX)
    r2 = jnp.float32(_RADIUS ** 2)

    def per_centroid(f, _):
        fv = jnp.full((16,), f, dtype=jnp.int32)
        cxb = plsc.load_gather(cxv, [fv])
        cyb = plsc.load_gather(cyv, [fv])
        czb = plsc.load_gather(czv, [fv])

        def insert_candidates(kv, gidx, pool):
            def w_cond(st):
                _, _, _, _, thr_c, rem = st
                return jnp.any((kv < thr_c) & rem)

            def w_step(st):
                pk0, pk1, pi0, pi1, _, rem = st
                cand = (kv < st[4]) & rem
                l = plsc.all_reduce_ffs(cand)
                onel = lanes == l
                ck = jnp.min(jnp.where(onel, kv, imax))
                ci = jnp.min(jnp.where(onel, gidx, imax))
                mk = jnp.maximum(jnp.max(pk0), jnp.max(pk1))
                mi = jnp.maximum(jnp.max(jnp.where(pk0 == mk, pi0, -1)),
                                 jnp.max(jnp.where(pk1 == mk, pi1, -1)))
                oh0 = (pk0 == mk) & (pi0 == mi)
                oh1 = (pk1 == mk) & (pi1 == mi)
                pk0 = jnp.where(oh0, ck, pk0)
                pk1 = jnp.where(oh1, ck, pk1)
                pi0 = jnp.where(oh0, ci, pi0)
                pi1 = jnp.where(oh1, ci, pi1)
                thr_n = jnp.maximum(jnp.max(pk0), jnp.max(pk1))
                return (pk0, pk1, pi0, pi1, thr_n, rem & (~onel))

            st = lax.while_loop(w_cond, w_step,
                                pool + (jnp.full((16,), True),))
            return st[:5]

        UNROLL = 8

        def scan_vreg(j, carry):
            pool = carry
            o = j * (16 * UNROLL)
            kvs = []
            for u in range(UNROLL):
                ou = o + u * 16
                dx = xv[pl.ds(ou, 16)] - cxb
                dy = yv[pl.ds(ou, 16)] - cyb
                dz = zv[pl.ds(ou, 16)] - czb
                dd = dx * dx + dy * dy + dz * dz
                kvs.append(jnp.where(dd <= r2, plsc.bitcast(dd, jnp.int32),
                                     _MBASE + (ou + lanes)))
            thr_c = pool[4]
            hit = kvs[0] < thr_c
            for u in range(1, UNROLL):
                hit = hit | (kvs[u] < thr_c)

            def do_insert(pool):
                for u in range(UNROLL):
                    pool = insert_candidates(kvs[u], o + u * 16 + lanes, pool)
                return pool

            return lax.cond(jnp.any(hit), do_insert, lambda p: p, pool)

        pool0 = (
            jnp.full((16,), imax, dtype=jnp.int32),
            jnp.full((16,), imax, dtype=jnp.int32),
            (1 << 24) + lanes,
            (1 << 24) + 16 + lanes,
            imax,
        )
        pk0, pk1, pi0, pi1, _ = lax.fori_loop(0, N // (16 * UNROLL),
                                              scan_vreg, pool0)

        def extract(k, st):
            pk0, pk1, pi0, pi1, iv0, iv1 = st
            mk = jnp.minimum(jnp.min(pk0), jnp.min(pk1))
            mi = jnp.minimum(jnp.min(jnp.where(pk0 == mk, pi0, imax)),
                             jnp.min(jnp.where(pk1 == mk, pi1, imax)))
            oh0 = (pk0 == mk) & (pi0 == mi)
            oh1 = (pk1 == mk) & (pi1 == mi)
            pk0 = jnp.where(oh0, imax, pk0)
            pk1 = jnp.where(oh1, imax, pk1)
            iv0 = jnp.where(lanes == k, mi, iv0)
            iv1 = jnp.where(lanes == (k - 16), mi, iv1)
            return (pk0, pk1, pi0, pi1, iv0, iv1)

        zero16 = jnp.zeros((16,), dtype=jnp.int32)
        _, _, _, _, iv0, iv1 = lax.fori_loop(
            0, _GROUP_NUM, extract, (pk0, pk1, pi0, pi1, zero16, zero16))

        base = f * _GROUP_NUM
        rxv[pl.ds(base, 16)] = plsc.load_gather(xv, [iv0]) - cxb
        ryv[pl.ds(base, 16)] = plsc.load_gather(yv, [iv0]) - cyb
        rzv[pl.ds(base, 16)] = plsc.load_gather(zv, [iv0]) - czb
        rxv[pl.ds(base + 16, 16)] = plsc.load_gather(xv, [iv1]) - cxb
        ryv[pl.ds(base + 16, 16)] = plsc.load_gather(yv, [iv1]) - cyb
        rzv[pl.ds(base + 16, 16)] = plsc.load_gather(zv, [iv1]) - czb
        return 0

    lax.fori_loop(0, CW, per_centroid, 0)
    pltpu.sync_copy(rxv, rx_ref.at[pl.ds(wid * GW, GW)])
    pltpu.sync_copy(ryv, ry_ref.at[pl.ds(wid * GW, GW)])
    pltpu.sync_copy(rzv, rz_ref.at[pl.ds(wid * GW, GW)])


@jax.jit
def kernel(xyz):
    B, N, _ = xyz.shape
    F, G, C = _FPS_NUM, _GROUP_NUM, _BQ_BLOCK
    xyz_t = jnp.transpose(xyz, (2, 0, 1))
    X, Y, Z = xyz_t[0], xyz_t[1], xyz_t[2]

    f0 = jax.random.randint(jax.random.key(1), (B,), 0, N, dtype=jnp.int32)
    c0 = xyz[jnp.arange(B), f0]
    c0x, c0y, c0z = c0[:, 0:1], c0[:, 1:2], c0[:, 2:3]

    cxs = jax.ShapeDtypeStruct((B, F), jnp.float32)
    CX, CY, CZ = pl.pallas_call(
        _fps_body,
        out_shape=(cxs, cxs, cxs),
        scratch_shapes=[pltpu.VMEM((B, N), jnp.float32)],
    )(X, Y, Z, c0x, c0y, c0z)

    mesh = plsc.VectorSubcoreMesh(core_axis_name="c", subcore_axis_name="s")
    GW = (F // 4) * G
    relf = jax.ShapeDtypeStruct((B * F * G,), jnp.float32)
    RX, RY, RZ = pl.kernel(
        _sc_bq_body,
        mesh=mesh,
        out_type=(relf, relf, relf),
        compiler_params=pltpu.CompilerParams(needs_layout_passes=False),
        scratch_types=[
            pltpu.VMEM((N,), jnp.float32),
            pltpu.VMEM((N,), jnp.float32),
            pltpu.VMEM((N,), jnp.float32),
            pltpu.VMEM((F // 4,), jnp.float32),
            pltpu.VMEM((F // 4,), jnp.float32),
            pltpu.VMEM((F // 4,), jnp.float32),
            pltpu.VMEM((GW,), jnp.float32),
            pltpu.VMEM((GW,), jnp.float32),
            pltpu.VMEM((GW,), jnp.float32),
        ],
    )(X, Y, Z, CX.reshape(-1), CY.reshape(-1), CZ.reshape(-1))

    cent = jnp.stack([CX, CY, CZ], axis=-1)
    rel = jnp.stack([RX.reshape(B, F, G), RY.reshape(B, F, G),
                     RZ.reshape(B, F, G)], axis=-1)
    combined = jnp.concatenate([cent[:, :, None, :], rel], axis=2)
    return (combined, cent)


# speedup vs baseline: 2.8283x; 1.0580x over previous
import functools

import jax
import jax.numpy as jnp
import numpy as np
from jax import lax
from jax.experimental import pallas as pl
from jax.experimental.pallas import tpu as pltpu
from jax.experimental.pallas import tpu_sc as plsc

_MBASE = int(np.float32(1e10).view(np.int32))
_IMAX = 2147483647

_FPS_NUM = 512
_GROUP_NUM = 32
_RADIUS = 0.2
_BQ_BLOCK = 128


def _fps_body(x_ref, y_ref, z_ref, c0x_ref, c0y_ref, c0z_ref,
              cx_ref, cy_ref, cz_ref, dist_ref):
    B, N = x_ref.shape
    dist_ref[...] = jnp.full((B, N), 1e10, dtype=jnp.float32)
    iota = jax.lax.broadcasted_iota(jnp.int32, (B, N), 1)
    col = jax.lax.broadcasted_iota(jnp.int32, cx_ref.shape, 1)

    def body(i, carry):
        cx, cy, cz = carry
        cx_ref[...] = jnp.where(col == i, cx, cx_ref[...])
        cy_ref[...] = jnp.where(col == i, cy, cy_ref[...])
        cz_ref[...] = jnp.where(col == i, cz, cz_ref[...])
        dx = x_ref[...] - cx
        dy = y_ref[...] - cy
        dz = z_ref[...] - cz
        dd = dx * dx + dy * dy + dz * dz
        dold = dist_ref[...]
        dnew = jnp.where(dd < dold, dd, dold)
        dist_ref[...] = dnew
        m = jnp.max(dnew, axis=1, keepdims=True)
        tie = jnp.where(dnew == m, iota, N)
        amin = jnp.min(tie, axis=1, keepdims=True)
        em = iota == amin
        ncx = jnp.sum(jnp.where(em, x_ref[...], 0.0), axis=1, keepdims=True)
        ncy = jnp.sum(jnp.where(em, y_ref[...], 0.0), axis=1, keepdims=True)
        ncz = jnp.sum(jnp.where(em, z_ref[...], 0.0), axis=1, keepdims=True)
        return (ncx, ncy, ncz)

    jax.lax.fori_loop(0, _FPS_NUM, body,
                      (c0x_ref[...], c0y_ref[...], c0z_ref[...]))


def _sc_bq_body(x_ref, y_ref, z_ref, cx_ref, cy_ref, cz_ref,
                rx_ref, ry_ref, rz_ref,
                xv, yv, zv, cxv, cyv, czv, rxv, ryv, rzv):
    NC = 2
    wid = lax.axis_index("s") * NC + lax.axis_index("c")
    b = wid // 4
    CW = 128
    GW = CW * _GROUP_NUM
    N = x_ref.shape[1]
    pltpu.sync_copy(x_ref.at[b], xv)
    pltpu.sync_copy(y_ref.at[b], yv)
    pltpu.sync_copy(z_ref.at[b], zv)
    pltpu.sync_copy(cx_ref.at[pl.ds(wid * CW, CW)], cxv)
    pltpu.sync_copy(cy_ref.at[pl.ds(wid * CW, CW)], cyv)
    pltpu.sync_copy(cz_ref.at[pl.ds(wid * CW, CW)], czv)

    lanes = lax.iota(jnp.int32, 16)
    imax = jnp.int32(_IMAX)
    r2 = jnp.float32(_RADIUS ** 2)

    def per_centroid(f, _):
        fv = jnp.full((16,), f, dtype=jnp.int32)
        cxb = plsc.load_gather(cxv, [fv])
        cyb = plsc.load_gather(cyv, [fv])
        czb = plsc.load_gather(czv, [fv])

        def insert_candidates(kv, gidx, pool):
            def w_cond(st):
                _, _, _, _, thr_c, rem = st
                return jnp.any((kv < thr_c) & rem)

            def w_step(st):
                pk0, pk1, pi0, pi1, _, rem = st
                cand = (kv < st[4]) & rem
                l = plsc.all_reduce_ffs(cand)
                onel = lanes == l
                ck = jnp.min(jnp.where(onel, kv, imax))
                ci = jnp.min(jnp.where(onel, gidx, imax))
                mk = jnp.maximum(jnp.max(pk0), jnp.max(pk1))
                mi = jnp.maximum(jnp.max(jnp.where(pk0 == mk, pi0, -1)),
                                 jnp.max(jnp.where(pk1 == mk, pi1, -1)))
                oh0 = (pk0 == mk) & (pi0 == mi)
                oh1 = (pk1 == mk) & (pi1 == mi)
                pk0 = jnp.where(oh0, ck, pk0)
                pk1 = jnp.where(oh1, ck, pk1)
                pi0 = jnp.where(oh0, ci, pi0)
                pi1 = jnp.where(oh1, ci, pi1)
                thr_n = jnp.maximum(jnp.max(pk0), jnp.max(pk1))
                return (pk0, pk1, pi0, pi1, thr_n, rem & (~onel))

            st = lax.while_loop(w_cond, w_step,
                                pool + (jnp.full((16,), True),))
            return st[:5]

        UNROLL = 16

        def scan_vreg(j, carry):
            pool = carry
            o = j * (16 * UNROLL)
            kvs = []
            for u in range(UNROLL):
                ou = o + u * 16
                dx = xv[pl.ds(ou, 16)] - cxb
                dy = yv[pl.ds(ou, 16)] - cyb
                dz = zv[pl.ds(ou, 16)] - czb
                dd = dx * dx + dy * dy + dz * dz
                kvs.append(jnp.where(dd <= r2, plsc.bitcast(dd, jnp.int32),
                                     _MBASE + (ou + lanes)))
            thr_c = pool[4]
            hit = kvs[0] < thr_c
            for u in range(1, UNROLL):
                hit = hit | (kvs[u] < thr_c)

            def do_insert(pool):
                for u in range(UNROLL):
                    pool = insert_candidates(kvs[u], o + u * 16 + lanes, pool)
                return pool

            return lax.cond(jnp.any(hit), do_insert, lambda p: p, pool)

        pool0 = (
            jnp.full((16,), imax, dtype=jnp.int32),
            jnp.full((16,), imax, dtype=jnp.int32),
            (1 << 24) + lanes,
            (1 << 24) + 16 + lanes,
            imax,
        )
        pk0, pk1, pi0, pi1, _ = lax.fori_loop(0, N // (16 * UNROLL),
                                              scan_vreg, pool0)

        def extract(k, st):
            pk0, pk1, pi0, pi1, iv0, iv1 = st
            mk = jnp.minimum(jnp.min(pk0), jnp.min(pk1))
            mi = jnp.minimum(jnp.min(jnp.where(pk0 == mk, pi0, imax)),
                             jnp.min(jnp.where(pk1 == mk, pi1, imax)))
            oh0 = (pk0 == mk) & (pi0 == mi)
            oh1 = (pk1 == mk) & (pi1 == mi)
            pk0 = jnp.where(oh0, imax, pk0)
            pk1 = jnp.where(oh1, imax, pk1)
            iv0 = jnp.where(lanes == k, mi, iv0)
            iv1 = jnp.where(lanes == (k - 16), mi, iv1)
            return (pk0, pk1, pi0, pi1, iv0, iv1)

        zero16 = jnp.zeros((16,), dtype=jnp.int32)
        _, _, _, _, iv0, iv1 = lax.fori_loop(
            0, _GROUP_NUM, extract, (pk0, pk1, pi0, pi1, zero16, zero16))

        base = f * _GROUP_NUM
        rxv[pl.ds(base, 16)] = plsc.load_gather(xv, [iv0]) - cxb
        ryv[pl.ds(base, 16)] = plsc.load_gather(yv, [iv0]) - cyb
        rzv[pl.ds(base, 16)] = plsc.load_gather(zv, [iv0]) - czb
        rxv[pl.ds(base + 16, 16)] = plsc.load_gather(xv, [iv1]) - cxb
        ryv[pl.ds(base + 16, 16)] = plsc.load_gather(yv, [iv1]) - cyb
        rzv[pl.ds(base + 16, 16)] = plsc.load_gather(zv, [iv1]) - czb
        return 0

    lax.fori_loop(0, CW, per_centroid, 0)
    pltpu.sync_copy(rxv, rx_ref.at[pl.ds(wid * GW, GW)])
    pltpu.sync_copy(ryv, ry_ref.at[pl.ds(wid * GW, GW)])
    pltpu.sync_copy(rzv, rz_ref.at[pl.ds(wid * GW, GW)])


@jax.jit
def kernel(xyz):
    B, N, _ = xyz.shape
    F, G, C = _FPS_NUM, _GROUP_NUM, _BQ_BLOCK
    xyz_t = jnp.transpose(xyz, (2, 0, 1))
    X, Y, Z = xyz_t[0], xyz_t[1], xyz_t[2]

    f0 = jax.random.randint(jax.random.key(1), (B,), 0, N, dtype=jnp.int32)
    c0 = xyz[jnp.arange(B), f0]
    c0x, c0y, c0z = c0[:, 0:1], c0[:, 1:2], c0[:, 2:3]

    cxs = jax.ShapeDtypeStruct((B, F), jnp.float32)
    CX, CY, CZ = pl.pallas_call(
        _fps_body,
        out_shape=(cxs, cxs, cxs),
        scratch_shapes=[pltpu.VMEM((B, N), jnp.float32)],
    )(X, Y, Z, c0x, c0y, c0z)

    mesh = plsc.VectorSubcoreMesh(core_axis_name="c", subcore_axis_name="s")
    GW = (F // 4) * G
    relf = jax.ShapeDtypeStruct((B * F * G,), jnp.float32)
    RX, RY, RZ = pl.kernel(
        _sc_bq_body,
        mesh=mesh,
        out_type=(relf, relf, relf),
        compiler_params=pltpu.CompilerParams(needs_layout_passes=False),
        scratch_types=[
            pltpu.VMEM((N,), jnp.float32),
            pltpu.VMEM((N,), jnp.float32),
            pltpu.VMEM((N,), jnp.float32),
            pltpu.VMEM((F // 4,), jnp.float32),
            pltpu.VMEM((F // 4,), jnp.float32),
            pltpu.VMEM((F // 4,), jnp.float32),
            pltpu.VMEM((GW,), jnp.float32),
            pltpu.VMEM((GW,), jnp.float32),
            pltpu.VMEM((GW,), jnp.float32),
        ],
    )(X, Y, Z, CX.reshape(-1), CY.reshape(-1), CZ.reshape(-1))

    cent = jnp.stack([CX, CY, CZ], axis=-1)
    rel = jnp.stack([RX.reshape(B, F, G), RY.reshape(B, F, G),
                     RZ.reshape(B, F, G)], axis=-1)
    combined = jnp.concatenate([cent[:, :, None, :], rel], axis=2)
    return (combined, cent)
